# Initial kernel scaffold; baseline (speedup 1.0000x reference)
import jax
import jax.numpy as jnp
from jax.experimental import pallas as pl


def kernel(node_embedding, edge_index, W1, b1, W2, b2, Wl, bl):
    raise NotImplementedError("write your pallas kernel here")


# speedup vs baseline: 13.3028x; 13.3028x over previous
import functools

import jax
import jax.numpy as jnp
from jax import lax
from jax.experimental import pallas as pl
from jax.experimental.pallas import tpu as pltpu
from jax.experimental.pallas import tpu_sc as plsc

N = 10000
E = 320000
D = 128
H1 = 128
H2 = 32
OUT = 32

NC = 2
NS = 16
NW = NC * NS
CH = 128
NCHUNK = 80
SLAB = 8
NSLAB = NCHUNK // SLAB
EPAD = NW * NCHUNK * CH
NA = N + 8
RQ = 624

_mesh = plsc.VectorSubcoreMesh(core_axis_name="c", subcore_axis_name="s",
                               num_cores=NC)


def _zero_fill(buf, rows, width):
    z = jnp.zeros((16,), jnp.float32)

    @pl.loop(0, rows)
    def _(i):
        @pl.loop(0, width, step=16)
        def _(j):
            buf.at[i, pl.ds(j, 16)][...] = z


def _zero_acc(acc, zsrc, sid):
    base = sid * RQ

    @pl.loop(0, 40)
    def _(k):
        @pl.when(base + k * 16 < N)
        def _():
            pltpu.sync_copy(zsrc, acc.at[pl.ds(base + k * 16, 16)])


def _write_out(acc, out_hbm, cid, sid):
    base = sid * RQ

    @pl.loop(0, 40)
    def _(k):
        @pl.when(base + k * 16 < N)
        def _():
            pltpu.sync_copy(acc.at[pl.ds(base + k * 16, 16)],
                            out_hbm.at[cid, pl.ds(base + k * 16, 16)])


@functools.partial(
    pl.kernel,
    out_type=jax.ShapeDtypeStruct((NC, N, 16), jnp.float32),
    mesh=_mesh,
    scratch_types=[
        pltpu.VMEM((NCHUNK, CH), jnp.int32),
        pltpu.VMEM((CH, 16), jnp.float32),
        pltpu.VMEM((16, 16), jnp.float32),
        pltpu.VMEM_SHARED((NA, 16), jnp.float32),
    ],
)
def _sc_hist(dst_hbm, out_hbm, dst_v, ones_v, zbuf, acc):
    cid = lax.axis_index("c")
    sid = lax.axis_index("s")
    wid = sid * NC + cid

    one = jnp.ones((16,), jnp.float32)

    @pl.loop(0, CH)
    def _(i):
        ones_v.at[i][...] = one

    _zero_fill(zbuf, 16, 16)
    _zero_acc(acc, zbuf, sid)
    plsc.subcore_barrier()

    pltpu.sync_copy(dst_hbm.at[wid], dst_v)

    @pl.loop(0, NCHUNK)
    def _(g):
        pltpu.sync_copy(ones_v, acc.at[dst_v.at[g]], add=True)

    plsc.subcore_barrier()
    _write_out(acc, out_hbm, cid, sid)


def _make_sc_agg(dm):
    @functools.partial(
        pl.kernel,
        out_type=jax.ShapeDtypeStruct((NC, N, dm), jnp.float32),
        mesh=_mesh,
        compiler_params=pltpu.CompilerParams(use_tc_tiling_on_sc=False),
        scratch_types=[
            pltpu.VMEM((SLAB, CH), jnp.int32),
            pltpu.VMEM((SLAB, CH), jnp.int32),
            pltpu.VMEM((SLAB, CH), jnp.int32),
            pltpu.VMEM((SLAB, CH), jnp.int32),
            pltpu.VMEM((CH, dm), jnp.float32),
            pltpu.VMEM((CH, dm), jnp.float32),
            pltpu.VMEM_SHARED((NA, dm), jnp.float32),
            pltpu.SemaphoreType.DMA,
            pltpu.SemaphoreType.DMA,
        ],
    )
    def agg(src_hbm, dst_hbm, h_hbm, out_hbm,
            sidx0, sidx1, didx0, didx1, rows0, rows1, acc, sem0, sem1):
        cid = lax.axis_index("c")
        sid = lax.axis_index("s")
        wid = sid * NC + cid

        sidx = (sidx0, sidx1)
        didx = (didx0, didx1)
        rows = (rows0, rows1)
        sems = (sem0, sem1)

        _zero_fill(rows0, 16, dm)
        _zero_acc(acc, rows0.at[pl.ds(0, 16)], sid)
        plsc.subcore_barrier()

        pltpu.sync_copy(src_hbm.at[wid, pl.ds(0, SLAB)], sidx[0])
        pltpu.sync_copy(dst_hbm.at[wid, pl.ds(0, SLAB)], didx[0])
        pltpu.async_copy(h_hbm.at[sidx[0].at[0]], rows0, sem0)
        pltpu.async_copy(h_hbm.at[sidx[0].at[1]], rows1, sem1)

        for c in range(NCHUNK):
            slab, off = divmod(c, SLAB)
            if off == 0 and slab + 1 < NSLAB:
                nb = (slab + 1) % 2
                pltpu.sync_copy(src_hbm.at[wid, pl.ds((slab + 1) * SLAB, SLAB)],
                                sidx[nb])
                pltpu.sync_copy(dst_hbm.at[wid, pl.ds((slab + 1) * SLAB, SLAB)],
                                didx[nb])
            b = c % 2
            pltpu.make_async_copy(h_hbm.at[sidx[0].at[0]], rows[b], sems[b]).wait()
            pltpu.sync_copy(rows[b], acc.at[didx[slab % 2].at[off]], add=True)
            n = c + 2
            if n < NCHUNK:
                nslab, noff = divmod(n, SLAB)
                pltpu.async_copy(h_hbm.at[sidx[nslab % 2].at[noff]],
                                 rows[b], sems[b])

        plsc.subcore_barrier()
        _write_out(acc, out_hbm, cid, sid)

    return agg


_sc_agg_d128 = _make_sc_agg(D)
_sc_agg_d32 = _make_sc_agg(H2)


def _tc_mm1(x, w):
    def body(x_ref, w_ref, o_ref):
        o_ref[...] = jnp.dot(x_ref[...], w_ref[...],
                             preferred_element_type=jnp.float32)

    return pl.pallas_call(
        body, out_shape=jax.ShapeDtypeStruct((N, H1), jnp.float32))(x, w)


def _dinv_col(degp):
    deg = 1.0 + degp[0] + degp[1]
    return 1.0 / jnp.sqrt(deg[:, :1])


def _tc_scale(h1, degp):
    def body(h_ref, d_ref, o_ref):
        o_ref[...] = h_ref[...] * _dinv_col(d_ref[...])

    return pl.pallas_call(
        body, out_shape=jax.ShapeDtypeStruct((N, H1), jnp.float32))(h1, degp)


def _tc_layer(s1p, h1p, degp, b1, w2):
    def body(s_ref, h_ref, d_ref, b_ref, w_ref, o_ref):
        dinv = _dinv_col(d_ref[...])
        y = jax.nn.relu((s_ref[0] + s_ref[1] + h_ref[...]) * dinv + b_ref[...])
        o_ref[...] = jnp.dot(y, w_ref[...],
                             preferred_element_type=jnp.float32) * dinv

    return pl.pallas_call(
        body, out_shape=jax.ShapeDtypeStruct((N, H2), jnp.float32))(
            s1p, h1p, degp, b1, w2)


def _tc_final(s2p, h2p, degp, b2, wl, bl):
    def body(s_ref, h_ref, d_ref, b_ref, w_ref, bl_ref, o_ref):
        dinv = _dinv_col(d_ref[...])
        y = jax.nn.relu((s_ref[0] + s_ref[1] + h_ref[...]) * dinv + b_ref[...])
        z = jnp.dot(y, w_ref[...], preferred_element_type=jnp.float32)
        z = z + bl_ref[...]
        o_ref[...] = jax.nn.softmax(z, axis=-1)

    return pl.pallas_call(
        body, out_shape=jax.ShapeDtypeStruct((N, OUT), jnp.float32))(
            s2p, h2p, degp, b2, wl, bl)


def kernel(node_embedding, edge_index, W1, b1, W2, b2, Wl, bl):
    pad = EPAD - E
    src = jnp.concatenate(
        [edge_index[0], jnp.zeros((pad,), jnp.int32)]).reshape(NW, NCHUNK, CH)
    dst = jnp.concatenate(
        [edge_index[1], jnp.full((pad,), N, jnp.int32)]).reshape(NW, NCHUNK, CH)
    b1 = b1.reshape(1, H1)
    b2 = b2.reshape(1, H2)
    bl = bl.reshape(1, OUT)

    degp = _sc_hist(dst)
    h1 = _tc_mm1(node_embedding, W1)
    h1p = _tc_scale(h1, degp)
    s1p = _sc_agg_d128(src, dst, h1p)
    h2p = _tc_layer(s1p, h1p, degp, b1, W2)
    s2p = _sc_agg_d32(src, dst, h2p)
    return _tc_final(s2p, h2p, degp, b2, Wl, bl)
